# Initial kernel scaffold; baseline (speedup 1.0000x reference)
import jax
import jax.numpy as jnp
from jax.experimental import pallas as pl


def kernel(triplets, embeddings):
    raise NotImplementedError("write your pallas kernel here")


# speedup vs baseline: 1.3454x; 1.3454x over previous
import functools

import jax
import jax.numpy as jnp
from jax import lax
from jax.experimental import pallas as pl
from jax.experimental.pallas import tpu as pltpu
from jax.experimental.pallas import tpu_sc as plsc

MARGIN_ = 0.2

B = 16384
D = 128
NW = 32
BPW = B // NW
C = 128
NCHUNK = BPW // C

_mesh = plsc.VectorSubcoreMesh(core_axis_name="c", subcore_axis_name="s")


@functools.partial(
    pl.kernel,
    mesh=_mesh,
    compiler_params=pltpu.CompilerParams(needs_layout_passes=False),
    out_type=[
        jax.ShapeDtypeStruct((B,), jnp.float32),
        jax.ShapeDtypeStruct((B,), jnp.float32),
    ],
    scratch_types=[
        pltpu.VMEM((C,), jnp.int32),
        pltpu.VMEM((C,), jnp.int32),
        pltpu.VMEM((C,), jnp.int32),
        pltpu.VMEM((C, D), jnp.float32),
        pltpu.VMEM((C, D), jnp.float32),
        pltpu.VMEM((C, D), jnp.float32),
        pltpu.VMEM((BPW,), jnp.float32),
        pltpu.VMEM((BPW,), jnp.float32),
        pltpu.SemaphoreType.DMA,
    ],
)
def _sc_dist2(emb_hbm, ia_hbm, ip_hbm, in_hbm, outp_hbm, outn_hbm,
              ia_v, ip_v, in_v, ra_v, rp_v, rn_v, op_v, on_v, sem):
    wid = lax.axis_index("s") * 2 + lax.axis_index("c")
    base = wid * BPW

    for c in range(NCHUNK):
        off = base + c * C
        pltpu.sync_copy(ia_hbm.at[pl.ds(off, C)], ia_v)
        pltpu.sync_copy(ip_hbm.at[pl.ds(off, C)], ip_v)
        pltpu.sync_copy(in_hbm.at[pl.ds(off, C)], in_v)
        cp_a = pltpu.async_copy(emb_hbm.at[ia_v], ra_v, sem)
        cp_p = pltpu.async_copy(emb_hbm.at[ip_v], rp_v, sem)
        cp_n = pltpu.async_copy(emb_hbm.at[in_v], rn_v, sem)
        cp_a.wait()
        cp_p.wait()
        cp_n.wait()

        lane = lax.iota(jnp.int32, 16)

        def body(g, _, c=c):
            vp = jnp.zeros((16,), jnp.float32)
            vn = jnp.zeros((16,), jnp.float32)
            for k in range(16):
                t = g * 16 + k
                acc_p = jnp.zeros((16,), jnp.float32)
                acc_n = jnp.zeros((16,), jnp.float32)
                for j in range(D // 16):
                    av = ra_v[t, pl.ds(j * 16, 16)]
                    pv = rp_v[t, pl.ds(j * 16, 16)]
                    nv = rn_v[t, pl.ds(j * 16, 16)]
                    dp = av - pv
                    dn = av - nv
                    acc_p = acc_p + dp * dp
                    acc_n = acc_n + dn * dn
                vp = jnp.where(lane == k, jnp.sum(acc_p), vp)
                vn = jnp.where(lane == k, jnp.sum(acc_n), vn)
            op_v[pl.ds(c * C + g * 16, 16)] = vp
            on_v[pl.ds(c * C + g * 16, 16)] = vn
            return 0

        lax.fori_loop(0, C // 16, body, 0)

    pltpu.sync_copy(op_v, outp_hbm.at[pl.ds(base, BPW)])
    pltpu.sync_copy(on_v, outn_hbm.at[pl.ds(base, BPW)])


def _tc_finish_body(pd2_ref, nd2_ref, out_ref):
    pd = jnp.sqrt(pd2_ref[...])
    nd = jnp.sqrt(nd2_ref[...])
    loss = jnp.maximum(pd - nd + MARGIN_, 0.0)
    out_ref[0, 0] = jnp.sum(loss) * (1.0 / B)


_tc_finish = pl.pallas_call(
    _tc_finish_body,
    out_shape=jax.ShapeDtypeStruct((1, 1), jnp.float32),
    out_specs=pl.BlockSpec(memory_space=pltpu.SMEM),
)


def kernel(triplets, embeddings):
    triplets = triplets.astype(jnp.int32)
    ia = triplets[:, 0]
    ip = triplets[:, 1]
    inn = triplets[:, 2]
    pd2, nd2 = _sc_dist2(embeddings, ia, ip, inn)
    out = _tc_finish(pd2.reshape(D, B // D), nd2.reshape(D, B // D))
    return out.reshape(())


# speedup vs baseline: 1.3811x; 1.0266x over previous
import functools

import jax
import jax.numpy as jnp
from jax import lax
from jax.experimental import pallas as pl
from jax.experimental.pallas import tpu as pltpu
from jax.experimental.pallas import tpu_sc as plsc

MARGIN_ = 0.2

B = 16384
D = 128
NW = 32
BPW = B // NW
C = 128
NCHUNK = BPW // C

_mesh = plsc.VectorSubcoreMesh(core_axis_name="c", subcore_axis_name="s")


def _sqrt16(x):
    xs = jnp.maximum(x, 1e-20)
    i = plsc.bitcast(xs, jnp.int32)
    i = 0x5F3759DF - lax.shift_right_logical(i, 1)
    y = plsc.bitcast(i, jnp.float32)
    for _ in range(3):
        y = y * (1.5 - 0.5 * xs * y * y)
    return x * y


@functools.partial(
    pl.kernel,
    mesh=_mesh,
    compiler_params=pltpu.CompilerParams(needs_layout_passes=False),
    out_type=jax.ShapeDtypeStruct((NW * 16,), jnp.float32),
    scratch_types=[
        pltpu.VMEM((3 * C,), jnp.int32),
        pltpu.VMEM((3 * C,), jnp.int32),
        pltpu.VMEM((C,), jnp.int32),
        pltpu.VMEM((C,), jnp.int32),
        pltpu.VMEM((C,), jnp.int32),
        pltpu.VMEM((C,), jnp.int32),
        pltpu.VMEM((C,), jnp.int32),
        pltpu.VMEM((C,), jnp.int32),
        pltpu.VMEM((C, D), jnp.float32),
        pltpu.VMEM((C, D), jnp.float32),
        pltpu.VMEM((C, D), jnp.float32),
        pltpu.VMEM((C, D), jnp.float32),
        pltpu.VMEM((C, D), jnp.float32),
        pltpu.VMEM((C, D), jnp.float32),
        pltpu.VMEM((16,), jnp.float32),
        pltpu.SemaphoreType.DMA,
        pltpu.SemaphoreType.DMA,
    ],
)
def _sc_loss(trip_hbm, emb_hbm, out_hbm,
             tb0, tb1, ia0, ia1, ip0, ip1, in0, in1,
             ra0, ra1, rp0, rp1, rn0, rn1, acc_v, sem0, sem1):
    wid = lax.axis_index("s") * 2 + lax.axis_index("c")
    base = wid * BPW
    tb = (tb0, tb1)
    ia = (ia0, ia1)
    ipx = (ip0, ip1)
    inx = (in0, in1)
    ra = (ra0, ra1)
    rp = (rp0, rp1)
    rn = (rn0, rn1)
    sems = (sem0, sem1)
    lane = lax.iota(jnp.int32, 16)
    lane3 = lane * 3

    def stage(c):
        b = c % 2
        off = (base + c * C) * 3
        pltpu.sync_copy(trip_hbm.at[pl.ds(off, 3 * C)], tb[b])
        for g in range(C // 16):
            gbase = lane3 + g * 48
            va = plsc.load_gather(tb[b], [gbase])
            vp = plsc.load_gather(tb[b], [gbase + 1])
            vn = plsc.load_gather(tb[b], [gbase + 2])
            ia[b][pl.ds(g * 16, 16)] = va
            ipx[b][pl.ds(g * 16, 16)] = vp
            inx[b][pl.ds(g * 16, 16)] = vn
        return (
            pltpu.async_copy(emb_hbm.at[ia[b]], ra[b], sems[b]),
            pltpu.async_copy(emb_hbm.at[ipx[b]], rp[b], sems[b]),
            pltpu.async_copy(emb_hbm.at[inx[b]], rn[b], sems[b]),
        )

    acc_v[...] = jnp.zeros((16,), jnp.float32)
    cps = stage(0)
    for c in range(NCHUNK):
        if c + 1 < NCHUNK:
            nxt = stage(c + 1)
        else:
            nxt = None
        for cp in cps:
            cp.wait()
        cps = nxt
        b = c % 2
        ra_b, rp_b, rn_b = ra[b], rp[b], rn[b]

        def body(g, _, ra_b=ra_b, rp_b=rp_b, rn_b=rn_b):
            vp2 = jnp.zeros((16,), jnp.float32)
            vn2 = jnp.zeros((16,), jnp.float32)
            for k in range(16):
                t = g * 16 + k
                acc_p = jnp.zeros((16,), jnp.float32)
                acc_n = jnp.zeros((16,), jnp.float32)
                for j in range(D // 16):
                    av = ra_b[t, pl.ds(j * 16, 16)]
                    pv = rp_b[t, pl.ds(j * 16, 16)]
                    nv = rn_b[t, pl.ds(j * 16, 16)]
                    dp = av - pv
                    dn = av - nv
                    acc_p = acc_p + dp * dp
                    acc_n = acc_n + dn * dn
                vp2 = jnp.where(lane == k, jnp.sum(acc_p), vp2)
                vn2 = jnp.where(lane == k, jnp.sum(acc_n), vn2)
            loss = jnp.maximum(_sqrt16(vp2) - _sqrt16(vn2) + MARGIN_, 0.0)
            acc_v[...] = acc_v[...] + loss
            return 0

        lax.fori_loop(0, C // 16, body, 0)

    pltpu.sync_copy(acc_v, out_hbm.at[pl.ds(wid * 16, 16)])


def _tc_finish_body(part_ref, out_ref):
    out_ref[0, 0] = jnp.sum(part_ref[...]) * (1.0 / B)


_tc_finish = pl.pallas_call(
    _tc_finish_body,
    out_shape=jax.ShapeDtypeStruct((1, 1), jnp.float32),
    out_specs=pl.BlockSpec(memory_space=pltpu.SMEM),
)


def kernel(triplets, embeddings):
    triplets = triplets.astype(jnp.int32)
    part = _sc_loss(triplets.reshape(-1), embeddings)
    return _tc_finish(part).reshape(())


# speedup vs baseline: 1.5661x; 1.1339x over previous
import functools

import jax
import jax.numpy as jnp
from jax import lax
from jax.experimental import pallas as pl
from jax.experimental.pallas import tpu as pltpu
from jax.experimental.pallas import tpu_sc as plsc

MARGIN_ = 0.2

B = 16384
D = 128
NW = 32
BPW = B // NW
C = 128
NCHUNK = BPW // C

_mesh = plsc.VectorSubcoreMesh(core_axis_name="c", subcore_axis_name="s")


def _sqrt16(x):
    xs = jnp.maximum(x, 1e-20)
    i = plsc.bitcast(xs, jnp.int32)
    i = 0x5F3759DF - lax.shift_right_logical(i, 1)
    y = plsc.bitcast(i, jnp.float32)
    for _ in range(3):
        y = y * (1.5 - 0.5 * xs * y * y)
    return x * y


@functools.partial(
    pl.kernel,
    mesh=_mesh,
    compiler_params=pltpu.CompilerParams(needs_layout_passes=False),
    out_type=jax.ShapeDtypeStruct((NW * 16,), jnp.float32),
    scratch_types=[
        pltpu.VMEM((C, 3), jnp.int32),
        pltpu.VMEM((C,), jnp.int32),
        pltpu.VMEM((C,), jnp.int32),
        pltpu.VMEM((C,), jnp.int32),
        pltpu.VMEM((C,), jnp.int32),
        pltpu.VMEM((C,), jnp.int32),
        pltpu.VMEM((C,), jnp.int32),
        pltpu.VMEM((C, D), jnp.float32),
        pltpu.VMEM((C, D), jnp.float32),
        pltpu.VMEM((C, D), jnp.float32),
        pltpu.VMEM((C, D), jnp.float32),
        pltpu.VMEM((C, D), jnp.float32),
        pltpu.VMEM((C, D), jnp.float32),
        pltpu.VMEM((16,), jnp.float32),
        pltpu.SemaphoreType.DMA,
        pltpu.SemaphoreType.DMA,
    ],
)
def _sc_loss(trip_hbm, emb_hbm, out_hbm,
             tb0, ia0, ia1, ip0, ip1, in0, in1,
             ra0, ra1, rp0, rp1, rn0, rn1, acc_v, sem0, sem1):
    wid = lax.axis_index("s") * 2 + lax.axis_index("c")
    base = wid * BPW
    tb = (tb0, tb0)
    ia = (ia0, ia1)
    ipx = (ip0, ip1)
    inx = (in0, in1)
    ra = (ra0, ra1)
    rp = (rp0, rp1)
    rn = (rn0, rn1)
    sems = (sem0, sem1)
    lane = lax.iota(jnp.int32, 16)

    def stage(c):
        b = c % 2
        off = base + c * C
        pltpu.sync_copy(trip_hbm.at[pl.ds(off, C), :], tb[b])
        zero = jnp.zeros((16,), jnp.int32)
        for g in range(C // 16):
            gbase = lane + g * 16
            va = plsc.load_gather(tb[b], [gbase, zero])
            vp = plsc.load_gather(tb[b], [gbase, zero + 1])
            vn = plsc.load_gather(tb[b], [gbase, zero + 2])
            ia[b][pl.ds(g * 16, 16)] = va
            ipx[b][pl.ds(g * 16, 16)] = vp
            inx[b][pl.ds(g * 16, 16)] = vn
        return (
            pltpu.async_copy(emb_hbm.at[ia[b]], ra[b], sems[b]),
            pltpu.async_copy(emb_hbm.at[ipx[b]], rp[b], sems[b]),
            pltpu.async_copy(emb_hbm.at[inx[b]], rn[b], sems[b]),
        )

    acc_v[...] = jnp.zeros((16,), jnp.float32)
    cps = stage(0)
    for c in range(NCHUNK):
        if c + 1 < NCHUNK:
            nxt = stage(c + 1)
        else:
            nxt = None
        for cp in cps:
            cp.wait()
        cps = nxt
        b = c % 2
        ra_b, rp_b, rn_b = ra[b], rp[b], rn[b]

        def body(g, _, ra_b=ra_b, rp_b=rp_b, rn_b=rn_b):
            vp2 = jnp.zeros((16,), jnp.float32)
            vn2 = jnp.zeros((16,), jnp.float32)
            for k in range(16):
                t = g * 16 + k
                acc_p = jnp.zeros((16,), jnp.float32)
                acc_n = jnp.zeros((16,), jnp.float32)
                for j in range(D // 16):
                    av = ra_b[t, pl.ds(j * 16, 16)]
                    pv = rp_b[t, pl.ds(j * 16, 16)]
                    nv = rn_b[t, pl.ds(j * 16, 16)]
                    dp = av - pv
                    dn = av - nv
                    acc_p = acc_p + dp * dp
                    acc_n = acc_n + dn * dn
                vp2 = jnp.where(lane == k, jnp.sum(acc_p), vp2)
                vn2 = jnp.where(lane == k, jnp.sum(acc_n), vn2)
            loss = jnp.maximum(_sqrt16(vp2) - _sqrt16(vn2) + MARGIN_, 0.0)
            acc_v[...] = acc_v[...] + loss
            return 0

        lax.fori_loop(0, C // 16, body, 0)

    pltpu.sync_copy(acc_v, out_hbm.at[pl.ds(wid * 16, 16)])


def _tc_finish_body(part_ref, out_ref):
    out_ref[0, 0] = jnp.sum(part_ref[...]) * (1.0 / B)


_tc_finish = pl.pallas_call(
    _tc_finish_body,
    out_shape=jax.ShapeDtypeStruct((1, 1), jnp.float32),
    out_specs=pl.BlockSpec(memory_space=pltpu.SMEM),
)


def kernel(triplets, embeddings):
    triplets = triplets.astype(jnp.int32)
    part = _sc_loss(triplets, embeddings)
    return _tc_finish(part).reshape(())
